# Initial kernel scaffold; baseline (speedup 1.0000x reference)
import jax
import jax.numpy as jnp
from jax.experimental import pallas as pl


def kernel(edge_index, rel_ids, e_table, r_table, W, b):
    raise NotImplementedError("write your pallas kernel here")


# speedup vs baseline: 1.8610x; 1.8610x over previous
import functools

import jax
import jax.numpy as jnp
from jax import lax
from jax.experimental import pallas as pl
from jax.experimental.pallas import tpu as pltpu
from jax.experimental.pallas import tpu_sc as plsc

V = 10000
R = 237
D = 200
E = 160000

NC = 2
NS = 16
NW = NC * NS

CHUNK = 200
CA, CB = 104, 96
NCHUNK = E // CHUNK
CPW = NCHUNK // NW
EPW = E // NW
VSTEPS = EPW // 16

ZROWS = 640
BN = 1000
H = D // 2
HP = 128

_mesh = plsc.VectorSubcoreMesh(core_axis_name="c", subcore_axis_name="s")
_sc_params = pltpu.CompilerParams(needs_layout_passes=False,
                                  use_tc_tiling_on_sc=False)


@functools.partial(
    pl.kernel,
    mesh=_mesh,
    out_type=(
        jax.ShapeDtypeStruct((V // BN, NW, BN), jnp.int32),
        jax.ShapeDtypeStruct((V // BN, NW, BN), jnp.int32),
        jax.ShapeDtypeStruct((E, D), jnp.float32),
    ),
    scratch_types=[
        pltpu.VMEM((EPW + 16,), jnp.int32),
        pltpu.VMEM((EPW + 16,), jnp.int32),
        pltpu.VMEM((V,), jnp.int32),
        pltpu.VMEM((V,), jnp.int32),
        pltpu.VMEM((CA,), jnp.int32),
        pltpu.VMEM((CB,), jnp.int32),
        pltpu.VMEM((CHUNK, D), jnp.float32),
        pltpu.SemaphoreType.DMA,
    ],
    compiler_params=_sc_params,
)
def _sc_deg_rel(src_hbm, dst_hbm, rel_hbm, rtab_hbm,
                degs_out, degd_out, rel_out,
                sidx, didx, hs, hd, ridx_a, ridx_b, rrows, sem):
    cid = lax.axis_index("c")
    sid = lax.axis_index("s")
    wid = sid * NC + cid

    zi = jnp.zeros((16,), jnp.int32)
    ones = jnp.ones((16,), jnp.int32)

    def _zb(i, carry):
        hs[pl.ds(i * 16, 16)] = zi
        hd[pl.ds(i * 16, 16)] = zi
        return carry
    lax.fori_loop(0, V // 16, _zb, 0)

    sidx[pl.ds(EPW - 8, 16)] = zi
    didx[pl.ds(EPW - 8, 16)] = zi

    def _ld(j, carry):
        base = (wid + j * NW) * CHUNK
        pltpu.sync_copy(src_hbm.at[pl.ds(base, CHUNK)], sidx.at[pl.ds(j * CHUNK, CHUNK)])
        pltpu.sync_copy(dst_hbm.at[pl.ds(base, CHUNK)], didx.at[pl.ds(j * CHUNK, CHUNK)])
        return carry
    lax.fori_loop(0, CPW, _ld, 0)

    def _hadd(hist, ids, elig):
        cnt, last = plsc.scan_count(ids, elig)
        plsc.addupdate_scatter(hist, [ids], cnt, mask=last)

    def _hb(i, carry):
        _hadd(hs, sidx[pl.ds(i * 16, 16)], None)
        _hadd(hd, didx[pl.ds(i * 16, 16)], None)
        return carry
    lax.fori_loop(0, VSTEPS, _hb, 0)
    tail = lax.iota(jnp.int32, 16) < 8
    _hadd(hs, sidx[pl.ds(VSTEPS * 16, 16)], tail)
    _hadd(hd, didx[pl.ds(VSTEPS * 16, 16)], tail)

    def _hw(k, carry):
        pltpu.sync_copy(hs.at[pl.ds(k * BN, BN)], degs_out.at[k, wid])
        pltpu.sync_copy(hd.at[pl.ds(k * BN, BN)], degd_out.at[k, wid])
        return carry
    lax.fori_loop(0, V // BN, _hw, 0)

    def _rg(j, carry):
        base = (wid + j * NW) * CHUNK
        pltpu.sync_copy(rel_hbm.at[pl.ds(base, CA)], ridx_a)
        pltpu.sync_copy(rel_hbm.at[pl.ds(base + CA, CB)], ridx_b)
        pltpu.async_copy(rtab_hbm.at[ridx_a], rrows.at[pl.ds(0, CA)], sem).wait()
        pltpu.async_copy(rtab_hbm.at[ridx_b], rrows.at[pl.ds(CA, CB)], sem).wait()
        pltpu.sync_copy(rrows, rel_out.at[pl.ds(base, CHUNK)])
        return carry
    lax.fori_loop(0, CPW, _rg, 0)


CPS = NCHUNK // NS


@functools.partial(
    pl.kernel,
    mesh=_mesh,
    out_type=jax.ShapeDtypeStruct((NC, V, HP), jnp.float32),
    scratch_types=[
        pltpu.VMEM_SHARED((V, HP), jnp.float32),
        pltpu.VMEM((CA,), jnp.int32),
        pltpu.VMEM((CB,), jnp.int32),
        pltpu.VMEM((CA,), jnp.int32),
        pltpu.VMEM((CB,), jnp.int32),
        pltpu.VMEM((CHUNK, HP), jnp.float32),
        pltpu.SemaphoreType.DMA,
    ],
    compiler_params=_sc_params,
)
def _sc_agg(srcb_hbm, dst_hbm, h_hbm, zeros_hbm,
            part_out,
            agg, sidx_a, sidx_b, didx_a, didx_b, rows, sem):
    cid = lax.axis_index("c")
    sid = lax.axis_index("s")

    @pl.when(sid < NS - 1)
    def _():
        pltpu.sync_copy(zeros_hbm, agg.at[pl.ds(sid * ZROWS, ZROWS)])

    @pl.when(sid == NS - 1)
    def _():
        pltpu.sync_copy(zeros_hbm.at[pl.ds(0, V - (NS - 1) * ZROWS)],
                        agg.at[pl.ds((NS - 1) * ZROWS, V - (NS - 1) * ZROWS)])

    plsc.subcore_barrier()

    def _body(j, carry):
        base = (sid + j * NS) * CHUNK
        sbase = cid * E + base
        pltpu.sync_copy(srcb_hbm.at[pl.ds(sbase, CA)], sidx_a)
        pltpu.sync_copy(srcb_hbm.at[pl.ds(sbase + CA, CB)], sidx_b)
        pltpu.sync_copy(dst_hbm.at[pl.ds(base, CA)], didx_a)
        pltpu.sync_copy(dst_hbm.at[pl.ds(base + CA, CB)], didx_b)
        pltpu.async_copy(h_hbm.at[sidx_a], rows.at[pl.ds(0, CA)], sem).wait()
        pltpu.async_copy(h_hbm.at[sidx_b], rows.at[pl.ds(CA, CB)], sem).wait()
        pltpu.sync_copy(rows.at[pl.ds(0, CA)], agg.at[didx_a], add=True)
        pltpu.sync_copy(rows.at[pl.ds(CA, CB)], agg.at[didx_b], add=True)
        return carry
    lax.fori_loop(0, CPS, _body, 0)

    plsc.subcore_barrier()

    @pl.when(sid < NS - 1)
    def _():
        pltpu.sync_copy(agg.at[pl.ds(sid * ZROWS, ZROWS)],
                        part_out.at[cid, pl.ds(sid * ZROWS, ZROWS)])

    @pl.when(sid == NS - 1)
    def _():
        pltpu.sync_copy(agg.at[pl.ds((NS - 1) * ZROWS, V - (NS - 1) * ZROWS)],
                        part_out.at[cid, pl.ds((NS - 1) * ZROWS, V - (NS - 1) * ZROWS)])


def _tc_h_body(pd_ref, e_ref, h_ref):
    deg = jnp.sum(pd_ref[0], axis=0).astype(jnp.float32)
    norm = lax.rsqrt(jnp.maximum(deg, 1.0))
    h_ref[...] = e_ref[...] * norm[:, None]


def _tc_out_body(p_ref, pd_ref, w_ref, b_ref, o_ref):
    deg = jnp.sum(pd_ref[0], axis=0).astype(jnp.float32)
    norm = lax.rsqrt(jnp.maximum(deg, 1.0))[:, None]
    o_ref[...] = (
        jnp.dot(p_ref[0] * norm, w_ref[0], preferred_element_type=jnp.float32)
        + jnp.dot(p_ref[1] * norm, w_ref[1], preferred_element_type=jnp.float32)
        + b_ref[...]
    )


def kernel(edge_index, rel_ids, e_table, r_table, W, b):
    src = edge_index[0]
    dst = edge_index[1]
    src_both = jnp.concatenate([src, src + V])
    e_cat = jnp.pad(
        e_table.reshape(V, NC, H).transpose(1, 0, 2), ((0, 0), (0, 0), (0, HP - H))
    ).reshape(NC * V, HP)
    w_pad = jnp.pad(W.reshape(NC, H, D), ((0, 0), (0, HP - H), (0, 0)))
    zeros_blk = jnp.zeros((ZROWS, HP), jnp.float32)

    degs, degd, rel_out = _sc_deg_rel(src, dst, rel_ids, r_table)

    h_cat = pl.pallas_call(
        _tc_h_body,
        grid=(NC * V // BN,),
        in_specs=[
            pl.BlockSpec((1, NW, BN), lambda i: (i % (V // BN), 0, 0)),
            pl.BlockSpec((BN, HP), lambda i: (i, 0)),
        ],
        out_specs=pl.BlockSpec((BN, HP), lambda i: (i, 0)),
        out_shape=jax.ShapeDtypeStruct((NC * V, HP), jnp.float32),
    )(degs, e_cat)

    part = _sc_agg(src_both, dst, h_cat, zeros_blk)

    node_out = pl.pallas_call(
        _tc_out_body,
        grid=(V // BN,),
        in_specs=[
            pl.BlockSpec((NC, BN, HP), lambda i: (0, i, 0)),
            pl.BlockSpec((1, NW, BN), lambda i: (i, 0, 0)),
            pl.BlockSpec((NC, HP, D), lambda i: (0, 0, 0)),
            pl.BlockSpec((1, D), lambda i: (0, 0)),
        ],
        out_specs=pl.BlockSpec((BN, D), lambda i: (i, 0)),
        out_shape=jax.ShapeDtypeStruct((V, D), jnp.float32),
    )(part, degd, w_pad, b.reshape(1, D))

    return node_out, rel_out
